# Initial kernel scaffold; baseline (speedup 1.0000x reference)
import functools
import math

import jax
import jax.numpy as jnp
from jax import lax
from jax.experimental import pallas as pl
from jax.experimental.pallas import tpu as pltpu
from jax.experimental.pallas import tpu_sc as plsc

NC = 2
NS = 16
NW = NC * NS
LANES = 16

SUB = 128
K = 8


def _make_emb_kernel(N0, D, scale):
    rows_per_w = N0 // NW
    nch = rows_per_w // K

    mesh = plsc.VectorSubcoreMesh(core_axis_name="c", subcore_axis_name="s")

    @functools.partial(
        pl.kernel,
        out_type=jax.ShapeDtypeStruct((N0, SUB, D), jnp.float32),
        mesh=mesh,
        scratch_types=[
            pltpu.VMEM((K, SUB), jnp.int32),
            pltpu.VMEM((K, SUB, D), jnp.float32),
            pltpu.SemaphoreType.DMA,
        ],
    )
    def emb(tokens_hbm, table_hbm, out_hbm, idx_v, rows_v, sem):
        wid = lax.axis_index("s") * NC + lax.axis_index("c")

        def chunk_body(ch, carry):
            rbase = wid * rows_per_w + ch * K
            pltpu.sync_copy(tokens_hbm.at[pl.ds(rbase, K)], idx_v)
            copies = [
                pltpu.async_copy(table_hbm.at[idx_v.at[j]], rows_v.at[j], sem)
                for j in range(K)
            ]
            for cp in copies:
                cp.wait()

            def scale_body(r, c2):
                for j in range(K):
                    for h in range(D // LANES):
                        sl = rows_v[j, r, pl.ds(h * LANES, LANES)]
                        rows_v[j, r, pl.ds(h * LANES, LANES)] = sl * scale
                return c2

            lax.fori_loop(0, SUB, scale_body, 0)
            pltpu.sync_copy(rows_v, out_hbm.at[pl.ds(rbase, K)])
            return carry

        lax.fori_loop(0, nch, chunk_body, 0)

    return emb


def kernel(tokens, table):
    B, L = tokens.shape
    V, D = table.shape
    N = B * L
    assert N % (NW * K * SUB) == 0 and D % LANES == 0
    N0 = N // SUB

    tok = tokens.astype(jnp.int32).reshape(N0, SUB)
    scale = math.sqrt(D)
    out = _make_emb_kernel(N0, D, scale)(tok, table)
    return out.reshape(B, L, D)


# speedup vs baseline: 1.4010x; 1.4010x over previous
import functools
import math

import jax
import jax.numpy as jnp
from jax import lax
from jax.experimental import pallas as pl
from jax.experimental.pallas import tpu as pltpu
from jax.experimental.pallas import tpu_sc as plsc

NC = 2
NS = 16
NW = NC * NS
LANES = 16

SUB = 128
K = 8


def _make_emb_kernel(N0, D, scale):
    rows_per_w = N0 // NW
    nch = rows_per_w // K

    mesh = plsc.VectorSubcoreMesh(core_axis_name="c", subcore_axis_name="s")

    @functools.partial(
        pl.kernel,
        out_type=jax.ShapeDtypeStruct((N0, SUB, D), jnp.float32),
        mesh=mesh,
        scratch_types=[
            pltpu.VMEM((K, SUB), jnp.int32),
            pltpu.VMEM((K, SUB, D), jnp.float32),
            pltpu.SemaphoreType.DMA,
        ],
        compiler_params=pltpu.CompilerParams(use_tc_tiling_on_sc=False),
    )
    def emb(tokens_hbm, table_hbm, out_hbm, idx_v, rows_v, sem):
        wid = lax.axis_index("s") * NC + lax.axis_index("c")

        def chunk_body(ch, carry):
            rbase = wid * rows_per_w + ch * K
            pltpu.sync_copy(tokens_hbm.at[pl.ds(rbase, K)], idx_v)
            copies = [
                pltpu.async_copy(table_hbm.at[idx_v.at[j]], rows_v.at[j], sem)
                for j in range(K)
            ]
            for cp in copies:
                cp.wait()

            def scale_body(r, c2):
                for j in range(K):
                    for h in range(D // LANES):
                        sl = rows_v[j, r, pl.ds(h * LANES, LANES)]
                        rows_v[j, r, pl.ds(h * LANES, LANES)] = sl * scale
                return c2

            lax.fori_loop(0, SUB, scale_body, 0)
            pltpu.sync_copy(rows_v, out_hbm.at[pl.ds(rbase, K)])
            return carry

        lax.fori_loop(0, nch, chunk_body, 0)

    return emb


def kernel(tokens, table):
    B, L = tokens.shape
    V, D = table.shape
    N = B * L
    assert N % (NW * K * SUB) == 0 and D % LANES == 0
    N0 = N // SUB

    tok = tokens.astype(jnp.int32).reshape(N0, SUB)
    scale = math.sqrt(D)
    out = _make_emb_kernel(N0, D, scale)(tok, table)
    return out.reshape(B, L, D)
